# baseline (speedup 1.0000x reference)
import functools

import jax
import jax.numpy as jnp
import numpy as np
from jax import lax
from jax.experimental import pallas as pl
from jax.experimental.pallas import tpu as pltpu
from jax.experimental.pallas import tpu_sc as plsc

_NUM_CLASSES = 100000
_DIM = 128
_BATCH = 16384
_NC, _NS, _L = 2, 16, 16
_NW = _NC * _NS
_ROWS_W = _BATCH // _NW
_CHUNK = 128
_NCHUNK = _ROWS_W // _CHUNK
_HIST_W = 6272
_HIST_PAD = _NS * _HIST_W
_Y_PER_TILE = _BATCH // _NS


def _sc_body(y, hidden, centers, part_out,
             hist, y1_v, ones2_v, y2_v, cnt_v, d_v, svec_v,
             cen0, cen1, hid0, hid1, zbuf,
             sem_h, sem_c0, sem_c1, sem_d0, sem_d1, sem_z):
  cid = lax.axis_index("c")
  sid = lax.axis_index("s")
  wid = sid * _NC + cid
  base = wid * _ROWS_W

  ydescs = [pltpu.async_copy(y.at[pl.ds(base + j * _CHUNK, _CHUNK)],
                             y2_v.at[j], sem_h) for j in range(_NCHUNK)]

  def _zfill(i, carry):
    zbuf[pl.ds(i * _L, _L)] = jnp.zeros((_L,), jnp.float32)
    return carry
  lax.fori_loop(0, _HIST_W // _L, _zfill, 0, unroll=4)
  for j in range(_Y_PER_TILE // _CHUNK):
    for q in range(_CHUNK // _L):
      ones2_v[j, pl.ds(q * _L, _L)] = jnp.ones((_L,), jnp.float32)

  zdesc = pltpu.make_async_copy(
      zbuf, hist.at[pl.ds(sid * _HIST_W, _HIST_W)], sem_z)
  zdesc.start()
  for dsc in ydescs:
    dsc.wait()

  cen_bufs = (cen0, cen1)
  hid_bufs = (hid0, hid1)
  sem_cs = (sem_c0, sem_c1)
  sem_ds = (sem_d0, sem_d1)

  def _start(j):
    b = j & 1
    cc = pltpu.async_copy(centers.at[y2_v.at[j]], cen_bufs[b], sem_cs[b])
    hh = pltpu.async_copy(
        hidden.at[pl.ds(base + j * _CHUNK, _CHUNK)], hid_bufs[b], sem_ds[b])
    return cc, hh

  pending = {0: _start(0)}

  sc1 = jax.named_scope("hist_phase"); sc1.__enter__()
  y1descs = [pltpu.async_copy(
      y.at[pl.ds(sid * _Y_PER_TILE + j * _CHUNK, _CHUNK)], y1_v.at[j], sem_h)
      for j in range(_Y_PER_TILE // _CHUNK)]
  for dsc in y1descs:
    dsc.wait()
  zdesc.wait()
  plsc.subcore_barrier()
  adds = [pltpu.make_async_copy(ones2_v.at[j], hist.at[y1_v.at[j]], sem_h)
          for j in range(_Y_PER_TILE // _CHUNK)]
  for dsc in adds:
    dsc.start(add=True)
  sc1.__exit__(None, None, None)

  sc3 = jax.named_scope("compute_phase"); sc3.__enter__()
  for j in range(_NCHUNK):
    b = j & 1
    cc, hh = pending.pop(j)
    if j + 1 < _NCHUNK:
      pending[j + 1] = _start(j + 1)
    cc.wait()
    hh.wait()
    cen = cen_bufs[b]
    hid = hid_bufs[b]

    def _row(r, carry, j=j, cen=cen, hid=hid):
      acc = jnp.zeros((_L,), jnp.float32)
      for q in range(_DIM // _L):
        h = hid[r, pl.ds(q * _L, _L)]
        c = cen[r, pl.ds(q * _L, _L)]
        dif = h - c
        acc = acc + dif * dif
      d_v[pl.ds((j * _CHUNK + r) * _L, _L)] = acc
      return carry
    lax.fori_loop(0, _CHUNK, _row, 0, unroll=2)
  sc3.__exit__(None, None, None)

  sc2 = jax.named_scope("weights_phase"); sc2.__enter__()
  for dsc in adds:
    dsc.wait()
  plsc.subcore_barrier()
  descs = [pltpu.async_copy(hist.at[y2_v.at[j]],
                            cnt_v.at[pl.ds(j * _CHUNK, _CHUNK)], sem_h)
           for j in range(_NCHUNK)]
  for dsc in descs:
    dsc.wait()

  def _fold(g, sv):
    cv = cnt_v[pl.ds(g * _L, _L)]
    wv = 1.0 / (cv + 1.0)
    for l in range(_L):
      sv = sv + d_v[pl.ds(g * (_L * _L) + l * _L, _L)] * wv[l]
    return sv
  svec = lax.fori_loop(0, _ROWS_W // _L, _fold,
                       jnp.zeros((_L,), jnp.float32))
  svec_v[...] = svec
  pltpu.sync_copy(svec_v, part_out.at[pl.ds(wid * _L, _L)])
  sc2.__exit__(None, None, None)


_sc_kernel = functools.partial(
    pl.kernel,
    out_type=jax.ShapeDtypeStruct((_NW * _L,), jnp.float32),
    mesh=plsc.VectorSubcoreMesh(core_axis_name="c", subcore_axis_name="s"),
    scratch_types=[
        pltpu.VMEM_SHARED((_HIST_PAD,), jnp.float32),
        pltpu.VMEM((_Y_PER_TILE // _CHUNK, _CHUNK), jnp.int32),
        pltpu.VMEM((_Y_PER_TILE // _CHUNK, _CHUNK), jnp.float32),
        pltpu.VMEM((_NCHUNK, _CHUNK), jnp.int32),
        pltpu.VMEM((_ROWS_W,), jnp.float32),
        pltpu.VMEM((_ROWS_W * _L,), jnp.float32),
        pltpu.VMEM((_L,), jnp.float32),
        pltpu.VMEM((_CHUNK, _DIM), jnp.float32),
        pltpu.VMEM((_CHUNK, _DIM), jnp.float32),
        pltpu.VMEM((_CHUNK, _DIM), jnp.float32),
        pltpu.VMEM((_CHUNK, _DIM), jnp.float32),
        pltpu.VMEM((_HIST_W,), jnp.float32),
        pltpu.SemaphoreType.DMA,
        pltpu.SemaphoreType.DMA,
        pltpu.SemaphoreType.DMA,
        pltpu.SemaphoreType.DMA,
        pltpu.SemaphoreType.DMA,
        pltpu.SemaphoreType.DMA,
    ],
)(_sc_body)


def _finish_body(p_ref, o_ref):
  s = jnp.sum(p_ref[...])
  o_ref[0, 0] = 0.5 * s / (s + 1e-6)


def kernel(y, hidden, centers):
  part = _sc_kernel(y, hidden, centers)
  out = pl.pallas_call(
      _finish_body,
      out_shape=jax.ShapeDtypeStruct((1, 1), jnp.float32),
      out_specs=pl.BlockSpec(memory_space=pltpu.SMEM),
  )(part)
  return out[0, 0]


# speedup vs baseline: 1.0700x; 1.0700x over previous
import functools

import jax
import jax.numpy as jnp
import numpy as np
from jax import lax
from jax.experimental import pallas as pl
from jax.experimental.pallas import tpu as pltpu
from jax.experimental.pallas import tpu_sc as plsc

_NUM_CLASSES = 100000
_DIM = 128
_BATCH = 16384
_NC, _NS, _L = 2, 16, 16
_NW = _NC * _NS
_ROWS_W = _BATCH // _NW
_CHUNK = 128
_NCHUNK = _ROWS_W // _CHUNK
_HIST_W = 6272
_HIST_PAD = _NS * _HIST_W
_Y_PER_TILE = _BATCH // _NS


def _sc_body(y, hidden, centers, part_out,
             hist, y1_v, ones2_v, y2_v, cnt_v, w_exp, svec_v,
             cen0, cen1, hid0, hid1, zbuf,
             sem_h, sem_c0, sem_c1, sem_d0, sem_d1, sem_z):
  cid = lax.axis_index("c")
  sid = lax.axis_index("s")
  wid = sid * _NC + cid
  base = wid * _ROWS_W

  ydescs = [pltpu.async_copy(y.at[pl.ds(base + j * _CHUNK, _CHUNK)],
                             y2_v.at[j], sem_h) for j in range(_NCHUNK)]

  def _zfill(i, carry):
    zbuf[pl.ds(i * _L, _L)] = jnp.zeros((_L,), jnp.float32)
    return carry
  lax.fori_loop(0, _HIST_W // _L, _zfill, 0, unroll=4)
  for j in range(_Y_PER_TILE // _CHUNK):
    for q in range(_CHUNK // _L):
      ones2_v[j, pl.ds(q * _L, _L)] = jnp.ones((_L,), jnp.float32)

  zdesc = pltpu.make_async_copy(
      zbuf, hist.at[pl.ds(sid * _HIST_W, _HIST_W)], sem_z)
  zdesc.start()
  for dsc in ydescs:
    dsc.wait()

  cen_bufs = (cen0, cen1)
  hid_bufs = (hid0, hid1)
  sem_cs = (sem_c0, sem_c1)
  sem_ds = (sem_d0, sem_d1)

  def _start(j):
    b = j & 1
    cc = pltpu.async_copy(centers.at[y2_v.at[j]], cen_bufs[b], sem_cs[b])
    hh = pltpu.async_copy(
        hidden.at[pl.ds(base + j * _CHUNK, _CHUNK)], hid_bufs[b], sem_ds[b])
    return cc, hh

  pending = {0: _start(0)}

  sc1 = jax.named_scope("hist_phase"); sc1.__enter__()
  y1descs = [pltpu.async_copy(
      y.at[pl.ds(sid * _Y_PER_TILE + j * _CHUNK, _CHUNK)], y1_v.at[j], sem_h)
      for j in range(_Y_PER_TILE // _CHUNK)]
  for dsc in y1descs:
    dsc.wait()
  zdesc.wait()
  plsc.subcore_barrier()
  adds = [pltpu.make_async_copy(ones2_v.at[j], hist.at[y1_v.at[j]], sem_h)
          for j in range(_Y_PER_TILE // _CHUNK)]
  for dsc in adds:
    dsc.start(add=True)
  for dsc in adds:
    dsc.wait()
  plsc.subcore_barrier()
  sc1.__exit__(None, None, None)

  sc2 = jax.named_scope("weights_phase"); sc2.__enter__()
  descs = [pltpu.async_copy(hist.at[y2_v.at[j]],
                            cnt_v.at[pl.ds(j * _CHUNK, _CHUNK)], sem_h)
           for j in range(_NCHUNK)]
  for dsc in descs:
    dsc.wait()
  ones = jnp.ones((_L,), jnp.float32)

  def _wexp(g, carry):
    cv = cnt_v[pl.ds(g * _L, _L)]
    wv = 1.0 / (cv + 1.0)
    bi = g * (_L * _L)
    for l in range(_L):
      w_exp[pl.ds(bi + l * _L, _L)] = ones * wv[l]
    return carry
  lax.fori_loop(0, _ROWS_W // _L, _wexp, 0)
  sc2.__exit__(None, None, None)

  sc3 = jax.named_scope("compute_phase"); sc3.__enter__()
  svec = jnp.zeros((_L,), jnp.float32)
  for j in range(_NCHUNK):
    b = j & 1
    cc, hh = pending.pop(j)
    if j + 1 < _NCHUNK:
      pending[j + 1] = _start(j + 1)
    cc.wait()
    hh.wait()
    cen = cen_bufs[b]
    hid = hid_bufs[b]

    def _row(r, sv, j=j, cen=cen, hid=hid):
      acc = jnp.zeros((_L,), jnp.float32)
      for q in range(_DIM // _L):
        h = hid[r, pl.ds(q * _L, _L)]
        c = cen[r, pl.ds(q * _L, _L)]
        dif = h - c
        acc = acc + dif * dif
      return sv + acc * w_exp[pl.ds((j * _CHUNK + r) * _L, _L)]
    svec = lax.fori_loop(0, _CHUNK, _row, svec, unroll=2)

  svec_v[...] = svec
  pltpu.sync_copy(svec_v, part_out.at[pl.ds(wid * _L, _L)])
  sc3.__exit__(None, None, None)


_sc_kernel = functools.partial(
    pl.kernel,
    out_type=jax.ShapeDtypeStruct((_NW * _L,), jnp.float32),
    mesh=plsc.VectorSubcoreMesh(core_axis_name="c", subcore_axis_name="s"),
    scratch_types=[
        pltpu.VMEM_SHARED((_HIST_PAD,), jnp.float32),
        pltpu.VMEM((_Y_PER_TILE // _CHUNK, _CHUNK), jnp.int32),
        pltpu.VMEM((_Y_PER_TILE // _CHUNK, _CHUNK), jnp.float32),
        pltpu.VMEM((_NCHUNK, _CHUNK), jnp.int32),
        pltpu.VMEM((_ROWS_W,), jnp.float32),
        pltpu.VMEM((_ROWS_W * _L,), jnp.float32),
        pltpu.VMEM((_L,), jnp.float32),
        pltpu.VMEM((_CHUNK, _DIM), jnp.float32),
        pltpu.VMEM((_CHUNK, _DIM), jnp.float32),
        pltpu.VMEM((_CHUNK, _DIM), jnp.float32),
        pltpu.VMEM((_CHUNK, _DIM), jnp.float32),
        pltpu.VMEM((_HIST_W,), jnp.float32),
        pltpu.SemaphoreType.DMA,
        pltpu.SemaphoreType.DMA,
        pltpu.SemaphoreType.DMA,
        pltpu.SemaphoreType.DMA,
        pltpu.SemaphoreType.DMA,
        pltpu.SemaphoreType.DMA,
    ],
)(_sc_body)


def _finish_body(p_ref, o_ref):
  s = jnp.sum(p_ref[...])
  o_ref[0, 0] = 0.5 * s / (s + 1e-6)


def kernel(y, hidden, centers):
  part = _sc_kernel(y, hidden, centers)
  out = pl.pallas_call(
      _finish_body,
      out_shape=jax.ShapeDtypeStruct((1, 1), jnp.float32),
      out_specs=pl.BlockSpec(memory_space=pltpu.SMEM),
  )(part)
  return out[0, 0]
